# speedup vs baseline: 2.9114x; 2.9114x over previous
import functools
import math

import jax
import jax.numpy as jnp
from jax import lax
from jax.experimental import pallas as pl
from jax.experimental.pallas import tpu as pltpu
from jax.experimental.pallas import tpu_sc as plsc

D_MODEL = 128
SCALE = math.sqrt(float(D_MODEL))
NUM_CORES = 2
NUM_SUBCORES = 16
NW = NUM_CORES * NUM_SUBCORES
B_TOTAL = 4096 * 50
BPW = B_TOTAL // NW
CHUNK = 400
NCHUNK = BPW // CHUNK
LANES = 16
VECS_PER_ROW = D_MODEL // LANES

_mesh = plsc.VectorSubcoreMesh(core_axis_name="c", subcore_axis_name="s")


@functools.partial(
    pl.kernel,
    out_type=jax.ShapeDtypeStruct((B_TOTAL, D_MODEL), jnp.float32),
    mesh=_mesh,
    scratch_types=[
        pltpu.VMEM((BPW,), jnp.int32),
        pltpu.VMEM((CHUNK, D_MODEL), jnp.float32),
        pltpu.VMEM((CHUNK, D_MODEL), jnp.float32),
        pltpu.SemaphoreType.DMA,
        pltpu.SemaphoreType.DMA,
    ],
)
def _emb_lookup(idx_hbm, table_hbm, out_hbm, idx_v, buf0, buf1, sem0, sem1):
    wid = lax.axis_index("s") * NUM_CORES + lax.axis_index("c")
    base = wid * BPW
    pltpu.sync_copy(idx_hbm.at[pl.ds(base, BPW)], idx_v)

    bufs = (buf0, buf1)
    sems = (sem0, sem1)

    pltpu.async_copy(table_hbm.at[idx_v.at[pl.ds(0, CHUNK)]], buf0, sem0)

    @pl.loop(0, NCHUNK, step=2)
    def _outer(g2):
        for b in range(2):
            g = g2 + b
            buf = bufs[b]
            @pl.when(g + 1 < NCHUNK)
            def _start_next():
                pltpu.async_copy(
                    table_hbm.at[idx_v.at[pl.ds((g + 1) * CHUNK, CHUNK)]],
                    bufs[1 - b],
                    sems[1 - b],
                )

            pltpu.make_async_copy(
                table_hbm.at[idx_v.at[pl.ds(g * CHUNK, CHUNK)]], buf, sems[b]
            ).wait()

            @pl.loop(0, CHUNK)
            def _scale(i):
                for j in range(VECS_PER_ROW):
                    sl = pl.ds(j * LANES, LANES)
                    buf[i, sl] = buf[i, sl] * SCALE

            pltpu.sync_copy(buf, out_hbm.at[pl.ds(base + g * CHUNK, CHUNK)])


def kernel(x, lut):
    idx = x.reshape(-1).astype(jnp.int32)
    out = _emb_lookup(idx, lut)
    return out.reshape(x.shape + (D_MODEL,))


# speedup vs baseline: 2.9202x; 1.0030x over previous
import functools
import math

import jax
import jax.numpy as jnp
from jax import lax
from jax.experimental import pallas as pl
from jax.experimental.pallas import tpu as pltpu
from jax.experimental.pallas import tpu_sc as plsc

D_MODEL = 128
SCALE = math.sqrt(float(D_MODEL))
NUM_CORES = 2
NUM_SUBCORES = 16
NW = NUM_CORES * NUM_SUBCORES
B_TOTAL = 4096 * 50
BPW = B_TOTAL // NW
CHUNK = 400
NCHUNK = BPW // CHUNK
LANES = 16
VECS_PER_ROW = D_MODEL // LANES

_mesh = plsc.VectorSubcoreMesh(core_axis_name="c", subcore_axis_name="s")


@functools.partial(
    pl.kernel,
    out_type=jax.ShapeDtypeStruct((B_TOTAL, D_MODEL), jnp.float32),
    mesh=_mesh,
    scratch_types=[
        pltpu.VMEM((BPW,), jnp.int32),
        pltpu.VMEM((CHUNK, D_MODEL), jnp.float32),
        pltpu.VMEM((CHUNK, D_MODEL), jnp.float32),
        pltpu.SemaphoreType.DMA,
        pltpu.SemaphoreType.DMA,
        pltpu.SemaphoreType.DMA,
        pltpu.SemaphoreType.DMA,
    ],
)
def _emb_lookup(idx_hbm, table_hbm, out_hbm, idx_v, buf0, buf1,
                in0, in1, ot0, ot1):
    wid = lax.axis_index("s") * NUM_CORES + lax.axis_index("c")
    base = wid * BPW
    pltpu.sync_copy(idx_hbm.at[pl.ds(base, BPW)], idx_v)

    bufs = (buf0, buf1)
    in_sems = (in0, in1)
    out_sems = (ot0, ot1)

    pltpu.async_copy(table_hbm.at[idx_v.at[pl.ds(0, CHUNK)]], buf0, in0)

    @pl.loop(0, NCHUNK, step=2)
    def _outer(g2):
        for b in range(2):
            g = g2 + b
            buf = bufs[b]
            nbuf = bufs[1 - b]

            @pl.when(g + 1 < NCHUNK)
            def _start_next():
                @pl.when(g >= 1)
                def _drain_prev_store():
                    pltpu.make_async_copy(
                        nbuf,
                        out_hbm.at[pl.ds(base + (g - 1) * CHUNK, CHUNK)],
                        out_sems[1 - b],
                    ).wait()

                pltpu.async_copy(
                    table_hbm.at[idx_v.at[pl.ds((g + 1) * CHUNK, CHUNK)]],
                    nbuf,
                    in_sems[1 - b],
                )

            pltpu.make_async_copy(
                table_hbm.at[idx_v.at[pl.ds(g * CHUNK, CHUNK)]], buf, in_sems[b]
            ).wait()

            @plsc.parallel_loop(0, CHUNK, unroll=4)
            def _scale(i):
                for j in range(VECS_PER_ROW):
                    sl = pl.ds(j * LANES, LANES)
                    buf[i, sl] = buf[i, sl] * SCALE

            pltpu.async_copy(
                buf, out_hbm.at[pl.ds(base + g * CHUNK, CHUNK)], out_sems[b]
            )

    pltpu.make_async_copy(
        buf0, out_hbm.at[pl.ds(base + (NCHUNK - 2) * CHUNK, CHUNK)], ot0
    ).wait()
    pltpu.make_async_copy(
        buf1, out_hbm.at[pl.ds(base + (NCHUNK - 1) * CHUNK, CHUNK)], ot1
    ).wait()


def kernel(x, lut):
    idx = x.reshape(-1).astype(jnp.int32)
    out = _emb_lookup(idx, lut)
    return out.reshape(x.shape + (D_MODEL,))


# speedup vs baseline: 5.2057x; 1.7826x over previous
import functools
import math

import jax
import jax.numpy as jnp
from jax import lax
from jax.experimental import pallas as pl
from jax.experimental.pallas import tpu as pltpu
from jax.experimental.pallas import tpu_sc as plsc

D_MODEL = 128
SCALE = math.sqrt(float(D_MODEL))
NUM_CORES = 2
NUM_SUBCORES = 16
NW = NUM_CORES * NUM_SUBCORES
N_I = 4096
N_J = 50
IPW = N_I // NW
BPW = IPW * N_J
GI = 4
ROWS = GI * N_J
NITER = IPW // GI
LANES = 16
VECS_PER_ROW = D_MODEL // LANES

_mesh = plsc.VectorSubcoreMesh(core_axis_name="c", subcore_axis_name="s")


@functools.partial(
    pl.kernel,
    out_type=jax.ShapeDtypeStruct((N_I, N_J, D_MODEL), jnp.float32),
    mesh=_mesh,
    compiler_params=pltpu.CompilerParams(use_tc_tiling_on_sc=True),
    scratch_types=[
        pltpu.VMEM((BPW,), jnp.int32),
        pltpu.VMEM((ROWS, D_MODEL), jnp.float32),
        pltpu.VMEM((ROWS, D_MODEL), jnp.float32),
        pltpu.VMEM((GI, N_J, D_MODEL), jnp.float32),
        pltpu.VMEM((GI, N_J, D_MODEL), jnp.float32),
        pltpu.SemaphoreType.DMA,
        pltpu.SemaphoreType.DMA,
        pltpu.SemaphoreType.DMA,
        pltpu.SemaphoreType.DMA,
    ],
)
def _emb_lookup(idx_hbm, table_hbm, out_hbm, idx_v, gbuf0, gbuf1,
                sbuf0, sbuf1, in0, in1, ot0, ot1):
    wid = lax.axis_index("s") * NUM_CORES + lax.axis_index("c")
    base = wid * BPW
    ibase = wid * IPW
    pltpu.sync_copy(idx_hbm.at[pl.ds(base, BPW)], idx_v)

    gbufs = (gbuf0, gbuf1)
    sbufs = (sbuf0, sbuf1)
    in_sems = (in0, in1)
    out_sems = (ot0, ot1)

    for b in range(2):
        pltpu.async_copy(
            table_hbm.at[idx_v.at[pl.ds(b * ROWS, ROWS)]], gbufs[b], in_sems[b]
        )

    @pl.loop(0, NITER, step=2)
    def _outer(t0):
        for b in range(2):
            t = t0 + b
            gbuf = gbufs[b]
            sbuf = sbufs[b]

            pltpu.make_async_copy(
                table_hbm.at[idx_v.at[pl.ds(t * ROWS, ROWS)]], gbuf, in_sems[b]
            ).wait()

            @pl.when(t >= 2)
            def _drain_prev_store():
                pltpu.make_async_copy(
                    sbuf,
                    out_hbm.at[pl.ds(ibase + (t - 2) * GI, GI)],
                    out_sems[b],
                ).wait()

            @plsc.parallel_loop(0, N_J, unroll=2)
            def _scale(j):
                for a in range(GI):
                    for k in range(VECS_PER_ROW):
                        sl = pl.ds(k * LANES, LANES)
                        sbuf[a, j, sl] = gbuf[a * N_J + j, sl] * SCALE

            pltpu.async_copy(
                sbuf, out_hbm.at[pl.ds(ibase + t * GI, GI)], out_sems[b]
            )

            @pl.when(t + 2 < NITER)
            def _start_next():
                pltpu.async_copy(
                    table_hbm.at[idx_v.at[pl.ds((t + 2) * ROWS, ROWS)]],
                    gbuf,
                    in_sems[b],
                )

    pltpu.make_async_copy(
        sbuf0, out_hbm.at[pl.ds(ibase + (NITER - 2) * GI, GI)], ot0
    ).wait()
    pltpu.make_async_copy(
        sbuf1, out_hbm.at[pl.ds(ibase + (NITER - 1) * GI, GI)], ot1
    ).wait()


def kernel(x, lut):
    idx = x.reshape(-1).astype(jnp.int32)
    return _emb_lookup(idx, lut)


# speedup vs baseline: 8.8708x; 1.7040x over previous
import functools
import math

import jax
import jax.numpy as jnp
from jax import lax
from jax.experimental import pallas as pl
from jax.experimental.pallas import tpu as pltpu
from jax.experimental.pallas import tpu_sc as plsc

D_MODEL = 128
SCALE = math.sqrt(float(D_MODEL))
NUM_CORES = 2
NUM_SUBCORES = 16
NW = NUM_CORES * NUM_SUBCORES
N_I = 4096
N_J = 50
B_TOTAL = N_I * N_J
BPW = B_TOTAL // NW
CHUNK = 400
NCHUNK = BPW // CHUNK
LANES = 16
VECS_PER_ROW = D_MODEL // LANES

_mesh = plsc.VectorSubcoreMesh(core_axis_name="c", subcore_axis_name="s")


@functools.partial(
    pl.kernel,
    out_type=jax.ShapeDtypeStruct((B_TOTAL, D_MODEL), jnp.float32),
    mesh=_mesh,
    compiler_params=pltpu.CompilerParams(use_tc_tiling_on_sc=True),
    scratch_types=[
        pltpu.VMEM((BPW,), jnp.int32),
        pltpu.VMEM((CHUNK, D_MODEL), jnp.float32),
        pltpu.VMEM((CHUNK, D_MODEL), jnp.float32),
        pltpu.SemaphoreType.DMA,
        pltpu.SemaphoreType.DMA,
        pltpu.SemaphoreType.DMA,
        pltpu.SemaphoreType.DMA,
    ],
)
def _emb_lookup(idx_hbm, table_hbm, out_hbm, idx_v, buf0, buf1,
                in0, in1, ot0, ot1):
    wid = lax.axis_index("s") * NUM_CORES + lax.axis_index("c")
    base = wid * BPW
    pltpu.sync_copy(idx_hbm.at[pl.ds(base, BPW)], idx_v)

    bufs = (buf0, buf1)
    in_sems = (in0, in1)
    out_sems = (ot0, ot1)

    pltpu.async_copy(table_hbm.at[idx_v.at[pl.ds(0, CHUNK)]], buf0, in0)

    @pl.loop(0, NCHUNK, step=2)
    def _outer(g2):
        for b in range(2):
            g = g2 + b
            buf = bufs[b]
            nbuf = bufs[1 - b]

            @pl.when(g + 1 < NCHUNK)
            def _start_next():
                @pl.when(g >= 1)
                def _drain_prev_store():
                    pltpu.make_async_copy(
                        nbuf,
                        out_hbm.at[pl.ds(base + (g - 1) * CHUNK, CHUNK)],
                        out_sems[1 - b],
                    ).wait()

                pltpu.async_copy(
                    table_hbm.at[idx_v.at[pl.ds((g + 1) * CHUNK, CHUNK)]],
                    nbuf,
                    in_sems[1 - b],
                )

            pltpu.make_async_copy(
                table_hbm.at[idx_v.at[pl.ds(g * CHUNK, CHUNK)]], buf, in_sems[b]
            ).wait()

            @plsc.parallel_loop(0, CHUNK, unroll=4)
            def _scale(i):
                for j in range(VECS_PER_ROW):
                    sl = pl.ds(j * LANES, LANES)
                    buf[i, sl] = buf[i, sl] * SCALE

            pltpu.async_copy(
                buf, out_hbm.at[pl.ds(base + g * CHUNK, CHUNK)], out_sems[b]
            )

    pltpu.make_async_copy(
        buf0, out_hbm.at[pl.ds(base + (NCHUNK - 2) * CHUNK, CHUNK)], ot0
    ).wait()
    pltpu.make_async_copy(
        buf1, out_hbm.at[pl.ds(base + (NCHUNK - 1) * CHUNK, CHUNK)], ot1
    ).wait()


def kernel(x, lut):
    idx = x.T.reshape(-1).astype(jnp.int32)
    out = _emb_lookup(idx, lut)
    return out.reshape(N_J, N_I, D_MODEL).transpose(1, 0, 2)


# speedup vs baseline: 8.9636x; 1.0105x over previous
import functools
import math

import jax
import jax.numpy as jnp
from jax import lax
from jax.experimental import pallas as pl
from jax.experimental.pallas import tpu as pltpu
from jax.experimental.pallas import tpu_sc as plsc

D_MODEL = 128
SCALE = math.sqrt(float(D_MODEL))
NUM_CORES = 2
NUM_SUBCORES = 16
NW = NUM_CORES * NUM_SUBCORES
N_I = 4096
N_J = 50
B_TOTAL = N_I * N_J
BPW = B_TOTAL // NW
CHUNK = 320
NCHUNK = BPW // CHUNK
NBUF = 3
MAIN = (NCHUNK // NBUF) * NBUF
LANES = 16
VECS_PER_ROW = D_MODEL // LANES

_mesh = plsc.VectorSubcoreMesh(core_axis_name="c", subcore_axis_name="s")


@functools.partial(
    pl.kernel,
    out_type=jax.ShapeDtypeStruct((B_TOTAL, D_MODEL), jnp.float32),
    mesh=_mesh,
    compiler_params=pltpu.CompilerParams(use_tc_tiling_on_sc=True),
    scratch_types=[
        pltpu.VMEM((BPW,), jnp.int32),
        pltpu.VMEM((CHUNK, D_MODEL), jnp.float32),
        pltpu.VMEM((CHUNK, D_MODEL), jnp.float32),
        pltpu.VMEM((CHUNK, D_MODEL), jnp.float32),
        pltpu.SemaphoreType.DMA,
        pltpu.SemaphoreType.DMA,
        pltpu.SemaphoreType.DMA,
        pltpu.SemaphoreType.DMA,
        pltpu.SemaphoreType.DMA,
        pltpu.SemaphoreType.DMA,
    ],
)
def _emb_lookup(idx_hbm, table_hbm, out_hbm, idx_v, buf0, buf1, buf2,
                in0, in1, in2, ot0, ot1, ot2):
    wid = lax.axis_index("s") * NUM_CORES + lax.axis_index("c")
    base = wid * BPW
    pltpu.sync_copy(idx_hbm.at[pl.ds(base, BPW)], idx_v)

    bufs = (buf0, buf1, buf2)
    in_sems = (in0, in1, in2)
    out_sems = (ot0, ot1, ot2)

    def start_gather(g, b):
        pltpu.async_copy(
            table_hbm.at[idx_v.at[pl.ds(g * CHUNK, CHUNK)]], bufs[b], in_sems[b]
        )

    def wait_gather(g, b):
        pltpu.make_async_copy(
            table_hbm.at[idx_v.at[pl.ds(g * CHUNK, CHUNK)]], bufs[b], in_sems[b]
        ).wait()

    def start_store(g, b):
        pltpu.async_copy(
            bufs[b], out_hbm.at[pl.ds(base + g * CHUNK, CHUNK)], out_sems[b]
        )

    def wait_store(g, b):
        pltpu.make_async_copy(
            bufs[b], out_hbm.at[pl.ds(base + g * CHUNK, CHUNK)], out_sems[b]
        ).wait()

    def scale(b):
        buf = bufs[b]

        @plsc.parallel_loop(0, CHUNK, unroll=4)
        def _scale(i):
            for j in range(VECS_PER_ROW):
                sl = pl.ds(j * LANES, LANES)
                buf[i, sl] = buf[i, sl] * SCALE

    def body(g, b):
        @pl.when(g + 2 < NCHUNK)
        def _prefetch():
            pb = (b + 2) % NBUF

            @pl.when(g >= 1)
            def _drain():
                wait_store(g - 1, pb)

            start_gather(g + 2, pb)

        wait_gather(g, b)
        scale(b)
        start_store(g, b)

    start_gather(0, 0)
    start_gather(1, 1)

    @pl.loop(0, MAIN, step=NBUF)
    def _outer(g0):
        for b in range(NBUF):
            body(g0 + b, b)

    for g in range(MAIN, NCHUNK):
        body(g, g % NBUF)

    for g in range(NCHUNK - NBUF, NCHUNK):
        wait_store(g, g % NBUF)


def kernel(x, lut):
    idx = x.T.reshape(-1).astype(jnp.int32)
    out = _emb_lookup(idx, lut)
    return out.reshape(N_J, N_I, D_MODEL).transpose(1, 0, 2)


# speedup vs baseline: 9.0936x; 1.0145x over previous
import functools
import math

import jax
import jax.numpy as jnp
from jax import lax
from jax.experimental import pallas as pl
from jax.experimental.pallas import tpu as pltpu
from jax.experimental.pallas import tpu_sc as plsc

D_MODEL = 128
SCALE = math.sqrt(float(D_MODEL))
NUM_CORES = 2
NUM_SUBCORES = 16
NW = NUM_CORES * NUM_SUBCORES
N_I = 4096
N_J = 50
B_TOTAL = N_I * N_J
BPW = B_TOTAL // NW
CHUNK = 200
NCHUNK = BPW // CHUNK
NBUF = 4
MAIN = (NCHUNK // NBUF) * NBUF
LANES = 16
VECS_PER_ROW = D_MODEL // LANES

_mesh = plsc.VectorSubcoreMesh(core_axis_name="c", subcore_axis_name="s")


@functools.partial(
    pl.kernel,
    out_type=jax.ShapeDtypeStruct((B_TOTAL, D_MODEL), jnp.float32),
    mesh=_mesh,
    compiler_params=pltpu.CompilerParams(use_tc_tiling_on_sc=True),
    scratch_types=[
        pltpu.VMEM((BPW,), jnp.int32),
        pltpu.VMEM((CHUNK, D_MODEL), jnp.float32),
        pltpu.VMEM((CHUNK, D_MODEL), jnp.float32),
        pltpu.VMEM((CHUNK, D_MODEL), jnp.float32),
        pltpu.VMEM((CHUNK, D_MODEL), jnp.float32),
    ] + [pltpu.SemaphoreType.DMA] * 8,
)
def _emb_lookup(idx_hbm, table_hbm, out_hbm, idx_v, buf0, buf1, buf2, buf3,
                in0, in1, in2, in3, ot0, ot1, ot2, ot3):
    wid = lax.axis_index("s") * NUM_CORES + lax.axis_index("c")
    base = wid * BPW
    pltpu.sync_copy(idx_hbm.at[pl.ds(base, BPW)], idx_v)

    bufs = (buf0, buf1, buf2, buf3)
    in_sems = (in0, in1, in2, in3)
    out_sems = (ot0, ot1, ot2, ot3)

    def start_gather(g, b):
        pltpu.async_copy(
            table_hbm.at[idx_v.at[pl.ds(g * CHUNK, CHUNK)]], bufs[b], in_sems[b]
        )

    def wait_gather(g, b):
        pltpu.make_async_copy(
            table_hbm.at[idx_v.at[pl.ds(g * CHUNK, CHUNK)]], bufs[b], in_sems[b]
        ).wait()

    def start_store(g, b):
        pltpu.async_copy(
            bufs[b], out_hbm.at[pl.ds(base + g * CHUNK, CHUNK)], out_sems[b]
        )

    def wait_store(g, b):
        pltpu.make_async_copy(
            bufs[b], out_hbm.at[pl.ds(base + g * CHUNK, CHUNK)], out_sems[b]
        ).wait()

    def scale(b):
        buf = bufs[b]

        @plsc.parallel_loop(0, CHUNK, unroll=4)
        def _scale(i):
            for j in range(VECS_PER_ROW):
                sl = pl.ds(j * LANES, LANES)
                buf[i, sl] = buf[i, sl] * SCALE

    def body(g, b):
        @pl.when(g + 3 < NCHUNK)
        def _prefetch():
            pb = (b + 3) % NBUF

            @pl.when(g >= 1)
            def _drain():
                wait_store(g - 1, pb)

            start_gather(g + 3, pb)

        wait_gather(g, b)
        scale(b)
        start_store(g, b)

    start_gather(0, 0)
    start_gather(1, 1)
    start_gather(2, 2)

    @pl.loop(0, MAIN, step=NBUF)
    def _outer(g0):
        for b in range(NBUF):
            body(g0 + b, b)

    for g in range(MAIN, NCHUNK):
        body(g, g % NBUF)

    for g in range(NCHUNK - NBUF, NCHUNK):
        wait_store(g, g % NBUF)


def kernel(x, lut):
    idx = x.T.reshape(-1).astype(jnp.int32)
    out = _emb_lookup(idx, lut)
    return out.reshape(N_J, N_I, D_MODEL).transpose(1, 0, 2)


# speedup vs baseline: 9.1354x; 1.0046x over previous
import functools
import math

import jax
import jax.numpy as jnp
from jax import lax
from jax.experimental import pallas as pl
from jax.experimental.pallas import tpu as pltpu
from jax.experimental.pallas import tpu_sc as plsc

D_MODEL = 128
SCALE = math.sqrt(float(D_MODEL))
NUM_CORES = 2
NUM_SUBCORES = 16
NW = NUM_CORES * NUM_SUBCORES
N_I = 4096
N_J = 50
B_TOTAL = N_I * N_J
BPW = B_TOTAL // NW
CHUNK = 160
NCHUNK = BPW // CHUNK
NBUF = 5
MAIN = (NCHUNK // NBUF) * NBUF
LANES = 16
VECS_PER_ROW = D_MODEL // LANES

_mesh = plsc.VectorSubcoreMesh(core_axis_name="c", subcore_axis_name="s")


@functools.partial(
    pl.kernel,
    out_type=jax.ShapeDtypeStruct((B_TOTAL, D_MODEL), jnp.float32),
    mesh=_mesh,
    compiler_params=pltpu.CompilerParams(use_tc_tiling_on_sc=True),
    scratch_types=[
        pltpu.VMEM((BPW,), jnp.int32),
    ] + [pltpu.VMEM((CHUNK, D_MODEL), jnp.float32)] * 5
      + [pltpu.SemaphoreType.DMA] * 10,
)
def _emb_lookup(idx_hbm, table_hbm, out_hbm, idx_v, buf0, buf1, buf2, buf3,
                buf4, in0, in1, in2, in3, in4, ot0, ot1, ot2, ot3, ot4):
    wid = lax.axis_index("s") * NUM_CORES + lax.axis_index("c")
    base = wid * BPW
    pltpu.sync_copy(idx_hbm.at[pl.ds(base, BPW)], idx_v)

    bufs = (buf0, buf1, buf2, buf3, buf4)
    in_sems = (in0, in1, in2, in3, in4)
    out_sems = (ot0, ot1, ot2, ot3, ot4)

    def start_gather(g, b):
        pltpu.async_copy(
            table_hbm.at[idx_v.at[pl.ds(g * CHUNK, CHUNK)]], bufs[b], in_sems[b]
        )

    def wait_gather(g, b):
        pltpu.make_async_copy(
            table_hbm.at[idx_v.at[pl.ds(g * CHUNK, CHUNK)]], bufs[b], in_sems[b]
        ).wait()

    def start_store(g, b):
        pltpu.async_copy(
            bufs[b], out_hbm.at[pl.ds(base + g * CHUNK, CHUNK)], out_sems[b]
        )

    def wait_store(g, b):
        pltpu.make_async_copy(
            bufs[b], out_hbm.at[pl.ds(base + g * CHUNK, CHUNK)], out_sems[b]
        ).wait()

    def scale(b):
        buf = bufs[b]

        @plsc.parallel_loop(0, CHUNK, unroll=4)
        def _scale(i):
            for j in range(VECS_PER_ROW):
                sl = pl.ds(j * LANES, LANES)
                buf[i, sl] = buf[i, sl] * SCALE

    def body(g, b):
        @pl.when(g + 4 < NCHUNK)
        def _prefetch():
            pb = (b + 4) % NBUF

            @pl.when(g >= 1)
            def _drain():
                wait_store(g - 1, pb)

            start_gather(g + 4, pb)

        wait_gather(g, b)
        scale(b)
        start_store(g, b)

    for _g in range(4):
        start_gather(_g, _g)

    @pl.loop(0, MAIN, step=NBUF)
    def _outer(g0):
        for b in range(NBUF):
            body(g0 + b, b)

    for g in range(MAIN, NCHUNK):
        body(g, g % NBUF)

    for g in range(NCHUNK - NBUF, NCHUNK):
        wait_store(g, g % NBUF)


def kernel(x, lut):
    idx = x.T.reshape(-1).astype(jnp.int32)
    out = _emb_lookup(idx, lut)
    return out.reshape(N_J, N_I, D_MODEL).transpose(1, 0, 2)


# speedup vs baseline: 9.1411x; 1.0006x over previous
import functools
import math

import jax
import jax.numpy as jnp
from jax import lax
from jax.experimental import pallas as pl
from jax.experimental.pallas import tpu as pltpu
from jax.experimental.pallas import tpu_sc as plsc

D_MODEL = 128
SCALE = math.sqrt(float(D_MODEL))
NUM_CORES = 2
NUM_SUBCORES = 16
NW = NUM_CORES * NUM_SUBCORES
N_I = 4096
N_J = 50
B_TOTAL = N_I * N_J
BPW = B_TOTAL // NW
CHUNK = 160
NCHUNK = BPW // CHUNK
NBUF = 5
MAIN = (NCHUNK // NBUF) * NBUF
LANES = 16
VECS_PER_ROW = D_MODEL // LANES

_mesh = plsc.VectorSubcoreMesh(core_axis_name="c", subcore_axis_name="s")


@functools.partial(
    pl.kernel,
    out_type=jax.ShapeDtypeStruct((B_TOTAL, D_MODEL), jnp.float32),
    mesh=_mesh,
    compiler_params=pltpu.CompilerParams(use_tc_tiling_on_sc=True, skip_device_barrier=True),
    scratch_types=[
        pltpu.VMEM((BPW,), jnp.int32),
    ] + [pltpu.VMEM((CHUNK, D_MODEL), jnp.float32)] * 5
      + [pltpu.SemaphoreType.DMA] * 10,
)
def _emb_lookup(idx_hbm, table_hbm, out_hbm, idx_v, buf0, buf1, buf2, buf3,
                buf4, in0, in1, in2, in3, in4, ot0, ot1, ot2, ot3, ot4):
    wid = lax.axis_index("s") * NUM_CORES + lax.axis_index("c")
    base = wid * BPW
    pltpu.sync_copy(idx_hbm.at[pl.ds(base, BPW)], idx_v)

    bufs = (buf0, buf1, buf2, buf3, buf4)
    in_sems = (in0, in1, in2, in3, in4)
    out_sems = (ot0, ot1, ot2, ot3, ot4)

    def start_gather(g, b):
        pltpu.async_copy(
            table_hbm.at[idx_v.at[pl.ds(g * CHUNK, CHUNK)]], bufs[b], in_sems[b]
        )

    def wait_gather(g, b):
        pltpu.make_async_copy(
            table_hbm.at[idx_v.at[pl.ds(g * CHUNK, CHUNK)]], bufs[b], in_sems[b]
        ).wait()

    def start_store(g, b):
        pltpu.async_copy(
            bufs[b], out_hbm.at[pl.ds(base + g * CHUNK, CHUNK)], out_sems[b]
        )

    def wait_store(g, b):
        pltpu.make_async_copy(
            bufs[b], out_hbm.at[pl.ds(base + g * CHUNK, CHUNK)], out_sems[b]
        ).wait()

    def scale(b):
        buf = bufs[b]

        @plsc.parallel_loop(0, CHUNK, unroll=4)
        def _scale(i):
            for j in range(VECS_PER_ROW):
                sl = pl.ds(j * LANES, LANES)
                buf[i, sl] = buf[i, sl] * SCALE

    def body(g, b):
        @pl.when(g + 4 < NCHUNK)
        def _prefetch():
            pb = (b + 4) % NBUF

            @pl.when(g >= 1)
            def _drain():
                wait_store(g - 1, pb)

            start_gather(g + 4, pb)

        wait_gather(g, b)
        scale(b)
        start_store(g, b)

    for _g in range(4):
        start_gather(_g, _g)

    @pl.loop(0, MAIN, step=NBUF)
    def _outer(g0):
        for b in range(NBUF):
            body(g0 + b, b)

    for g in range(MAIN, NCHUNK):
        body(g, g % NBUF)

    for g in range(NCHUNK - NBUF, NCHUNK):
        wait_store(g, g % NBUF)


def kernel(x, lut):
    idx = x.T.reshape(-1).astype(jnp.int32)
    out = _emb_lookup(idx, lut)
    return out.reshape(N_J, N_I, D_MODEL).transpose(1, 0, 2)


# speedup vs baseline: 9.1641x; 1.0025x over previous
import functools
import math

import jax
import jax.numpy as jnp
from jax import lax
from jax.experimental import pallas as pl
from jax.experimental.pallas import tpu as pltpu
from jax.experimental.pallas import tpu_sc as plsc

D_MODEL = 128
SCALE = math.sqrt(float(D_MODEL))
NUM_CORES = 2
NUM_SUBCORES = 16
NW = NUM_CORES * NUM_SUBCORES
N_I = 4096
N_J = 50
B_TOTAL = N_I * N_J
BPW = B_TOTAL // NW
CHUNK = 128
NCHUNK = BPW // CHUNK
NBUF = 6
MAIN = (NCHUNK // NBUF) * NBUF
LANES = 16
VECS_PER_ROW = D_MODEL // LANES

_mesh = plsc.VectorSubcoreMesh(core_axis_name="c", subcore_axis_name="s")


@functools.partial(
    pl.kernel,
    out_type=jax.ShapeDtypeStruct((B_TOTAL, D_MODEL), jnp.float32),
    mesh=_mesh,
    compiler_params=pltpu.CompilerParams(use_tc_tiling_on_sc=True),
    scratch_types=[
        pltpu.VMEM((BPW,), jnp.int32),
    ] + [pltpu.VMEM((CHUNK, D_MODEL), jnp.float32)] * 6
      + [pltpu.SemaphoreType.DMA] * 12,
)
def _emb_lookup(idx_hbm, table_hbm, out_hbm, idx_v, buf0, buf1, buf2, buf3,
                buf4, buf5, in0, in1, in2, in3, in4, in5,
                ot0, ot1, ot2, ot3, ot4, ot5):
    wid = lax.axis_index("s") * NUM_CORES + lax.axis_index("c")
    base = wid * BPW
    pltpu.sync_copy(idx_hbm.at[pl.ds(base, BPW)], idx_v)

    bufs = (buf0, buf1, buf2, buf3, buf4, buf5)
    in_sems = (in0, in1, in2, in3, in4, in5)
    out_sems = (ot0, ot1, ot2, ot3, ot4, ot5)

    def start_gather(g, b):
        pltpu.async_copy(
            table_hbm.at[idx_v.at[pl.ds(g * CHUNK, CHUNK)]], bufs[b], in_sems[b]
        )

    def wait_gather(g, b):
        pltpu.make_async_copy(
            table_hbm.at[idx_v.at[pl.ds(g * CHUNK, CHUNK)]], bufs[b], in_sems[b]
        ).wait()

    def start_store(g, b):
        pltpu.async_copy(
            bufs[b], out_hbm.at[pl.ds(base + g * CHUNK, CHUNK)], out_sems[b]
        )

    def wait_store(g, b):
        pltpu.make_async_copy(
            bufs[b], out_hbm.at[pl.ds(base + g * CHUNK, CHUNK)], out_sems[b]
        ).wait()

    def scale(b):
        buf = bufs[b]

        @plsc.parallel_loop(0, CHUNK, unroll=4)
        def _scale(i):
            for j in range(VECS_PER_ROW):
                sl = pl.ds(j * LANES, LANES)
                buf[i, sl] = buf[i, sl] * SCALE

    def body(g, b):
        @pl.when(g + 5 < NCHUNK)
        def _prefetch():
            pb = (b + 5) % NBUF

            @pl.when(g >= 1)
            def _drain():
                wait_store(g - 1, pb)

            start_gather(g + 5, pb)

        wait_gather(g, b)
        scale(b)
        start_store(g, b)

    for _g in range(5):
        start_gather(_g, _g)

    @pl.loop(0, MAIN, step=NBUF)
    def _outer(g0):
        for b in range(NBUF):
            body(g0 + b, b)

    for g in range(MAIN, NCHUNK):
        body(g, g % NBUF)

    for g in range(NCHUNK - NBUF, NCHUNK):
        wait_store(g, g % NBUF)


def kernel(x, lut):
    idx = x.T.reshape(-1).astype(jnp.int32)
    out = _emb_lookup(idx, lut)
    return out.reshape(N_J, N_I, D_MODEL).transpose(1, 0, 2)
